# baseline (speedup 1.0000x reference)
import jax
import jax.numpy as jnp
from jax import lax
from jax.experimental import pallas as pl
from jax.experimental.pallas import tpu as pltpu
from jax.experimental.pallas import tpu_sc as plsc

N_NODES = 10000
N_EDGES = 320000
D_IN = 128
N_HEADS = 8
D_OUT = 16

NC = 2
NS = 16
NW = NC * NS
BLK = 128
N_BLOCKS = N_EDGES // BLK
BPW = N_BLOCKS // NW
N_EXTRA = N_BLOCKS - BPW * NW
CBLK = 13
N_CHUNKS = BPW // CBLK
CE = CBLK * BLK


def _tc_body(x_ref, w_ref, as_ref, at_ref, o_ref):
    xb = x_ref[...]
    w = w_ref[...]
    jj = lax.broadcasted_iota(jnp.int32, (D_IN, 2 * N_HEADS), 0) >> 4
    hh = lax.broadcasted_iota(jnp.int32, (D_IN, 2 * N_HEADS), 1) & 7
    sel = (jj == hh).astype(jnp.float32)
    acol = jnp.concatenate(
        [jnp.broadcast_to(as_ref[...], (D_IN, N_HEADS)),
         jnp.broadcast_to(at_ref[...], (D_IN, N_HEADS))],
        axis=1)
    B = lax.dot_general(
        w, sel * acol, (((0,), (0,)), ((), ())),
        preferred_element_type=jnp.float32,
        precision=lax.Precision.HIGHEST,
    )
    S = lax.dot_general(
        xb, B, (((1,), (0,)), ((), ())),
        preferred_element_type=jnp.float32,
        precision=lax.Precision.HIGHEST,
    )
    S = -S
    u = lax.bitcast_convert_type(S, jnp.uint32) + jnp.uint32(0x8000)
    u1 = u[:, 0:N_HEADS] >> jnp.uint32(16)
    u2 = u[:, N_HEADS:2 * N_HEADS] & jnp.uint32(0xFFFF0000)
    o_ref[...] = lax.bitcast_convert_type(u1 | u2, jnp.int32)


def _tc_split_body(ei_ref, s_ref, t_ref):
    s_ref[...] = lax.shift_left(ei_ref[0, :], 3)
    t_ref[...] = lax.shift_left(ei_ref[1, :], 3)


_tc_split = pl.pallas_call(
    _tc_split_body,
    out_shape=[
        jax.ShapeDtypeStruct((N_EDGES,), jnp.int32),
        jax.ShapeDtypeStruct((N_EDGES,), jnp.int32),
    ],
)


_TC_ROWS = 1000
_tc_pack = pl.pallas_call(
    _tc_body,
    grid=(N_NODES // _TC_ROWS,),
    in_specs=[
        pl.BlockSpec((_TC_ROWS, D_IN), lambda i: (i, 0)),
        pl.BlockSpec((D_IN, D_IN), lambda i: (0, 0)),
        pl.BlockSpec((D_IN, 1), lambda i: (0, 0)),
        pl.BlockSpec((D_IN, 1), lambda i: (0, 0)),
    ],
    out_specs=pl.BlockSpec((_TC_ROWS, N_HEADS), lambda i: (i, 0)),
    out_shape=jax.ShapeDtypeStruct((N_NODES, N_HEADS), jnp.int32),
)


def _sc_body(tpack_hbm, src_hbm, trg_hbm, out_hbm, table_v,
             is0, it0, is1, it1, ov0, ov1, si0, si1, so0, so1):
    wid = lax.axis_index("s") * NC + lax.axis_index("c")
    base_block = wid * BPW

    starts = [base_block + c * CBLK for c in range(N_CHUNKS)] + [N_BLOCKS - CBLK]
    NTOT = len(starts)
    ibufs = [(is0, it0), (is1, it1)]
    obufs = [ov0, ov1]
    isems = [si0, si1]
    osems = [so0, so1]

    def start_in(c):
        s, t = ibufs[c % 2]
        gb0 = starts[c]
        return (pltpu.async_copy(src_hbm.at[pl.ds(gb0 * BLK, CE)], s, isems[c % 2]),
                pltpu.async_copy(trg_hbm.at[pl.ds(gb0 * BLK, CE)], t, isems[c % 2]))

    def compute(c):
        idx_s, idx_t = ibufs[c % 2]
        out_v = obufs[c % 2]

        @plsc.parallel_loop(0, CBLK * 8, unroll=6)
        def _group_loop(g):
            sva = idx_s[pl.ds(g * 16, 16)]
            tva = idx_t[pl.ds(g * 16, 16)]
            off = lax.shift_left(lax.shift_right_logical(g, 3), 10) \
                + lax.shift_left(g & 7, 4)
            for h in range(N_HEADS):
                a = plsc.load_gather(table_v, [sva + h if h else sva])
                b = plsc.load_gather(table_v, [tva + h if h else tva])
                sa = plsc.bitcast(lax.shift_left(a, 16), jnp.float32)
                sb = plsc.bitcast(b & jnp.int32(-65536), jnp.float32)
                y = 1.0 + jnp.exp(sa + sb)
                r = plsc.bitcast(jnp.int32(0x7EF477D5) - plsc.bitcast(y, jnp.int32),
                                 jnp.float32)
                r = r * (2.0 - y * r)
                out_v[pl.ds(off + h * BLK, 16)] = r

    pend_in = start_in(0)
    pltpu.sync_copy(tpack_hbm, table_v)
    pend_out = [None, None]
    for c in range(NTOT):
        next_in = start_in(c + 1) if c + 1 < NTOT else None
        pend_in[0].wait()
        pend_in[1].wait()
        if pend_out[c % 2] is not None:
            pend_out[c % 2].wait()
        compute(c)
        pend_out[c % 2] = pltpu.async_copy(
            obufs[c % 2],
            out_hbm.at[pl.ds(starts[c] * BLK * N_HEADS, CE * N_HEADS)],
            osems[c % 2])
        pend_in = next_in
    pend_out[(NTOT - 1) % 2].wait()
    pend_out[NTOT % 2].wait()


_sc_edge = pl.kernel(
    _sc_body,
    out_type=jax.ShapeDtypeStruct((N_EDGES * N_HEADS,), jnp.float32),
    mesh=plsc.VectorSubcoreMesh(core_axis_name="c", subcore_axis_name="s"),
    compiler_params=pltpu.CompilerParams(needs_layout_passes=False),
    scratch_types=[
        pltpu.VMEM((N_NODES * N_HEADS,), jnp.int32),
        pltpu.VMEM((CE,), jnp.int32),
        pltpu.VMEM((CE,), jnp.int32),
        pltpu.VMEM((CE,), jnp.int32),
        pltpu.VMEM((CE,), jnp.int32),
        pltpu.VMEM((CE * N_HEADS,), jnp.float32),
        pltpu.VMEM((CE * N_HEADS,), jnp.float32),
        pltpu.SemaphoreType.DMA,
        pltpu.SemaphoreType.DMA,
        pltpu.SemaphoreType.DMA,
        pltpu.SemaphoreType.DMA,
    ],
)


def kernel(x, edge_index, W, a_src, a_tgt):
    ei = edge_index.astype(jnp.int32)
    a_s = a_src.reshape(N_HEADS * D_OUT, 1)
    a_t = a_tgt.reshape(N_HEADS * D_OUT, 1)
    src, trg = _tc_split(ei)
    tpack = _tc_pack(x, W, a_s, a_t)
    out_flat = _sc_edge(tpack.reshape(-1), src, trg)
    return (out_flat.reshape(N_BLOCKS, N_HEADS, BLK)
            .swapaxes(1, 2)
            .reshape(N_EDGES, N_HEADS))


# speedup vs baseline: 1.1233x; 1.1233x over previous
import jax
import jax.numpy as jnp
from jax import lax
from jax.experimental import pallas as pl
from jax.experimental.pallas import tpu as pltpu
from jax.experimental.pallas import tpu_sc as plsc

N_NODES = 10000
N_EDGES = 320000
D_IN = 128
N_HEADS = 8
D_OUT = 16

NC = 2
NS = 16
NW = NC * NS
BLK = 128
N_BLOCKS = N_EDGES // BLK
BPW = N_BLOCKS // NW
N_EXTRA = N_BLOCKS - BPW * NW
CBLK = 13
N_CHUNKS = BPW // CBLK
CE = CBLK * BLK


def _tc_body(x_ref, w_ref, as_ref, at_ref, o_ref):
    xb = x_ref[...]
    w = w_ref[...]
    jj = lax.broadcasted_iota(jnp.int32, (D_IN, 2 * N_HEADS), 0) >> 4
    hh = lax.broadcasted_iota(jnp.int32, (D_IN, 2 * N_HEADS), 1) & 7
    sel = (jj == hh).astype(jnp.float32)
    acol = jnp.concatenate(
        [jnp.broadcast_to(as_ref[...], (D_IN, N_HEADS)),
         jnp.broadcast_to(at_ref[...], (D_IN, N_HEADS))],
        axis=1)
    B = lax.dot_general(
        w, sel * acol, (((0,), (0,)), ((), ())),
        preferred_element_type=jnp.float32,
        precision=lax.Precision.HIGHEST,
    )
    S = lax.dot_general(
        xb, B, (((1,), (0,)), ((), ())),
        preferred_element_type=jnp.float32,
        precision=lax.Precision.HIGHEST,
    )
    S = -S
    u = lax.bitcast_convert_type(S, jnp.uint32) + jnp.uint32(0x8000)
    u1 = u[:, 0:N_HEADS] >> jnp.uint32(16)
    u2 = u[:, N_HEADS:2 * N_HEADS] & jnp.uint32(0xFFFF0000)
    o_ref[...] = lax.bitcast_convert_type(u1 | u2, jnp.int32)


def _tc_split_body(ei_ref, s_ref, t_ref):
    s_ref[...] = lax.shift_left(ei_ref[0, :], 3)
    t_ref[...] = lax.shift_left(ei_ref[1, :], 3)


_tc_split = pl.pallas_call(
    _tc_split_body,
    out_shape=[
        jax.ShapeDtypeStruct((N_EDGES,), jnp.int32),
        jax.ShapeDtypeStruct((N_EDGES,), jnp.int32),
    ],
)


_TC_ROWS = 1000
_tc_pack = pl.pallas_call(
    _tc_body,
    grid=(N_NODES // _TC_ROWS,),
    in_specs=[
        pl.BlockSpec((_TC_ROWS, D_IN), lambda i: (i, 0)),
        pl.BlockSpec((D_IN, D_IN), lambda i: (0, 0)),
        pl.BlockSpec((D_IN, 1), lambda i: (0, 0)),
        pl.BlockSpec((D_IN, 1), lambda i: (0, 0)),
    ],
    out_specs=pl.BlockSpec((_TC_ROWS, N_HEADS), lambda i: (i, 0)),
    out_shape=jax.ShapeDtypeStruct((N_NODES, N_HEADS), jnp.int32),
)


def _sc_body(tpack_hbm, src_hbm, trg_hbm, out_hbm, table_v,
             is0, it0, is1, it1, ov0, ov1, si0, si1, so0, so1):
    wid = lax.axis_index("s") * NC + lax.axis_index("c")
    base_block = wid * BPW

    starts = [base_block + c * CBLK for c in range(N_CHUNKS)] + [N_BLOCKS - CBLK]
    NTOT = len(starts)
    ibufs = [(is0, it0), (is1, it1)]
    obufs = [ov0, ov1]
    isems = [si0, si1]
    osems = [so0, so1]

    def start_in(c):
        s, t = ibufs[c % 2]
        gb0 = starts[c]
        return (pltpu.async_copy(src_hbm.at[pl.ds(gb0 * BLK, CE)], s, isems[c % 2]),
                pltpu.async_copy(trg_hbm.at[pl.ds(gb0 * BLK, CE)], t, isems[c % 2]))

    def compute(c):
        idx_s, idx_t = ibufs[c % 2]
        out_v = obufs[c % 2]

        @plsc.parallel_loop(0, CBLK * 8, unroll=4)
        def _group_loop(g):
            sva = idx_s[pl.ds(g * 16, 16)]
            tva = idx_t[pl.ds(g * 16, 16)]
            off = lax.shift_left(lax.shift_right_logical(g, 3), 10) \
                + lax.shift_left(g & 7, 4)
            for h in range(N_HEADS):
                a = plsc.load_gather(table_v, [sva + h if h else sva])
                b = plsc.load_gather(table_v, [tva + h if h else tva])
                sa = plsc.bitcast(lax.shift_left(a, 16), jnp.float32)
                sb = plsc.bitcast(b & jnp.int32(-65536), jnp.float32)
                y = 1.0 + jnp.exp(sa + sb)
                r = plsc.bitcast(jnp.int32(0x7EF477D5) - plsc.bitcast(y, jnp.int32),
                                 jnp.float32)
                r = r * (2.0 - y * r)
                out_v[pl.ds(off + h * BLK, 16)] = r

    pend_in = start_in(0)
    pltpu.sync_copy(tpack_hbm, table_v)
    pend_out = [None, None]
    for c in range(NTOT):
        next_in = start_in(c + 1) if c + 1 < NTOT else None
        pend_in[0].wait()
        pend_in[1].wait()
        if pend_out[c % 2] is not None:
            pend_out[c % 2].wait()
        compute(c)
        pend_out[c % 2] = pltpu.async_copy(
            obufs[c % 2],
            out_hbm.at[pl.ds(starts[c] * BLK * N_HEADS, CE * N_HEADS)],
            osems[c % 2])
        pend_in = next_in
    pend_out[(NTOT - 1) % 2].wait()
    pend_out[NTOT % 2].wait()


_sc_edge = pl.kernel(
    _sc_body,
    out_type=jax.ShapeDtypeStruct((N_EDGES * N_HEADS,), jnp.float32),
    mesh=plsc.VectorSubcoreMesh(core_axis_name="c", subcore_axis_name="s"),
    compiler_params=pltpu.CompilerParams(needs_layout_passes=False),
    scratch_types=[
        pltpu.VMEM((N_NODES * N_HEADS,), jnp.int32),
        pltpu.VMEM((CE,), jnp.int32),
        pltpu.VMEM((CE,), jnp.int32),
        pltpu.VMEM((CE,), jnp.int32),
        pltpu.VMEM((CE,), jnp.int32),
        pltpu.VMEM((CE * N_HEADS,), jnp.float32),
        pltpu.VMEM((CE * N_HEADS,), jnp.float32),
        pltpu.SemaphoreType.DMA,
        pltpu.SemaphoreType.DMA,
        pltpu.SemaphoreType.DMA,
        pltpu.SemaphoreType.DMA,
    ],
)


def kernel(x, edge_index, W, a_src, a_tgt):
    ei = edge_index.astype(jnp.int32)
    a_s = a_src.reshape(N_HEADS * D_OUT, 1)
    a_t = a_tgt.reshape(N_HEADS * D_OUT, 1)
    src, trg = _tc_split(ei)
    tpack = _tc_pack(x, W, a_s, a_t)
    out_flat = _sc_edge(tpack.reshape(-1), src, trg)
    return (out_flat.reshape(N_BLOCKS, N_HEADS, BLK)
            .swapaxes(1, 2)
            .reshape(N_EDGES, N_HEADS))
